# baseline (device time: 22399 ns/iter reference)
import jax
import jax.numpy as jnp
from jax import lax
from jax.experimental import pallas as pl
from jax.experimental.pallas import tpu as pltpu

CK = 32
EXTRA = 2


def kernel(x):
    m, n = x.shape
    half = m // 2
    hc = half // CK
    cy = hc + EXTRA
    cx = hc - EXTRA

    def body(x_ref, out_ref, xbf_ref, recv_ref,
             y_send_sems, y_recv_sems, x_send_sems, x_recv_sems):
        my_x = lax.axis_index("x")
        my_y = lax.axis_index("y")
        my_z = lax.axis_index("z")
        y_nbr = (my_x, 1 - my_y, my_z)
        x_nbr = (1 - my_x, my_y, my_z)

        def start_y(c, xv):
            return xv * (m - CK) + (1 - 2 * xv) * c * CK

        barrier = pltpu.get_barrier_semaphore()
        for nbr in (y_nbr, x_nbr):
            pl.semaphore_signal(
                barrier, inc=1, device_id=nbr,
                device_id_type=pl.DeviceIdType.MESH,
            )
        pl.semaphore_wait(barrier, 2)

        rdma_y = []
        for c in range(cy):
            rows = pl.ds(start_y(c, my_x), CK)
            xbf_ref[pl.ds(c * CK, CK), :] = x_ref[rows, :].astype(jnp.bfloat16)
            r = pltpu.make_async_remote_copy(
                src_ref=xbf_ref.at[pl.ds(c * CK, CK), :],
                dst_ref=recv_ref.at[rows, :],
                send_sem=y_send_sems.at[c],
                recv_sem=y_recv_sems.at[c],
                device_id=y_nbr,
                device_id_type=pl.DeviceIdType.MESH,
            )
            r.start()
            rdma_y.append(r)

        rdma_x = []
        for c in range(cy):
            rdma_y[c].wait_recv()
            rows = pl.ds(start_y(c, my_x), CK)
            if c < cx:
                r = pltpu.make_async_remote_copy(
                    src_ref=recv_ref.at[rows, :],
                    dst_ref=recv_ref.at[rows, :],
                    send_sem=x_send_sems.at[c],
                    recv_sem=x_recv_sems.at[c],
                    device_id=x_nbr,
                    device_id_type=pl.DeviceIdType.MESH,
                )
                r.start()
                rdma_x.append(r)
            out_ref[rows, :] = (
                x_ref[rows, :] + recv_ref[rows, :].astype(jnp.float32)
            )

        for c in range(cx):
            rdma_x[c].wait_recv()
            rows = pl.ds(start_y(c, 1 - my_x), CK)
            out_ref[rows, :] = (
                x_ref[rows, :] + recv_ref[rows, :].astype(jnp.float32)
            )

        for c in range(cy):
            rdma_y[c].wait_send()
        for c in range(cx):
            rdma_x[c].wait_send()

    return pl.pallas_call(
        body,
        out_shape=jax.ShapeDtypeStruct((m, n), jnp.float32),
        in_specs=[pl.BlockSpec(memory_space=pltpu.VMEM)],
        out_specs=pl.BlockSpec(memory_space=pltpu.VMEM),
        scratch_shapes=[
            pltpu.VMEM((cy * CK, n), jnp.bfloat16),
            pltpu.VMEM((m, n), jnp.bfloat16),
            pltpu.SemaphoreType.DMA((cy,)),
            pltpu.SemaphoreType.DMA((cy,)),
            pltpu.SemaphoreType.DMA((cx,)),
            pltpu.SemaphoreType.DMA((cx,)),
        ],
        compiler_params=pltpu.CompilerParams(collective_id=0),
    )(x)
